# baseline (speedup 1.0000x reference)
import jax
import jax.numpy as jnp
from jax.experimental import pallas as pl
from jax.experimental.pallas import tpu as pltpu

N = 262144
C_IN = 6
C = 256
NUM_CLS = 20

TILE_A = 32768
TILE = 4096

_f32 = jnp.float32
_HIGH = jax.lax.Precision.HIGHEST


def _aug(x6, rows):
    lane = jax.lax.broadcasted_iota(jnp.int32, (rows, 8), 1)
    x8 = jnp.pad(x6, ((0, 0), (0, 2)))
    return jnp.where(lane == 6, 1.0, x8)


def _gram_kernel(x_ref, out_ref):
    i = pl.program_id(0)
    x = _aug(x_ref[...], TILE_A)
    g = jax.lax.dot_general(x, x, (((0,), (0,)), ((), ())),
                            preferred_element_type=_f32,
                            precision=_HIGH)
    g = jnp.pad(g, ((0, 0), (0, 120)))

    @pl.when(i == 0)
    def _():
        out_ref[...] = jnp.zeros_like(out_ref)

    out_ref[...] += g


def _loss_kernel(x_ref, diff_src_ref, seg_ref, inst_ref, s_ref,
                 wbb_ref, bbb_ref, w1_ref, gbb_ref, eye_ref,
                 w2_ref, b2_ref, ws_ref, bs_ref, out_ref, acc_ref, ab_ref):
    i = pl.program_id(0)
    nsteps = pl.num_programs(0)

    def dot(a, b):
        return jax.lax.dot_general(a, b, (((1,), (0,)), ((), ())),
                                   preferred_element_type=_f32)

    def dot_t(a, b):
        return jax.lax.dot_general(a, b, (((1,), (1,)), ((), ())),
                                   preferred_element_type=_f32)

    @pl.when(i == 0)
    def _():
        acc_ref[...] = jnp.zeros_like(acc_ref)
        gam = gbb_ref[:, 0:1]
        bet = gbb_ref[:, 1:2]
        b1c = gbb_ref[:, 2:3]
        col = jax.lax.broadcasted_iota(jnp.int32, (C, 8), 1)
        w_base = wbb_ref[...] + jnp.where(col == 6, bbb_ref[...], 0.0)
        w_aug = dot(w1_ref[...], w_base) + jnp.where(col == 6, b1c, 0.0)
        s8 = s_ref[:, 0:8]
        nf = _f32(N)
        mu = dot(w_aug, s8[:, 6:7]) / nf
        sw = dot(w_aug, s8)
        ex2 = jnp.sum(sw * w_aug, axis=1, keepdims=True) / nf
        var = ex2 - mu * mu
        sd = jnp.sqrt(var + 1e-3)
        a_col = gam / sd
        b_col = (b1c - mu) * a_col + bet
        ab_ref[:, 0:1] = a_col
        ab_ref[:, 1:2] = b_col

    x = x_ref[...]
    feat = dot_t(wbb_ref[:, 0:6], x) + bbb_ref[...]
    h = dot(w1_ref[...], feat)
    hn = jnp.maximum(h * ab_ref[:, 0:1] + ab_ref[:, 1:2], 0.0)
    bp = dot(w2_ref[...], hn) + b2_ref[...]
    bp3 = bp[0:3, :]

    diff = diff_src_ref[...]
    d3 = diff[:, 3:6] - diff[:, 0:3]
    bgt = dot_t(eye_ref[...], d3)
    bg3 = bgt[0:3, :]

    inst = inst_ref[...].reshape(1, TILE)
    mask = (inst != -1).astype(_f32)

    bias_dist = jnp.sum(jnp.abs(bp3 - bg3), axis=0, keepdims=True)
    bp_n = jnp.sqrt(jnp.sum(bp3 * bp3, axis=0, keepdims=True))
    bg_n = jnp.sqrt(jnp.sum(bg3 * bg3, axis=0, keepdims=True))
    bpn = bp3 / (bp_n + 1e-8)
    bgn = bg3 / (bg_n + 1e-8)
    cos = -jnp.sum(bpn * bgn, axis=0, keepdims=True)

    logit = dot(ws_ref[...], feat) + bs_ref[...]
    seg = seg_ref[...].reshape(1, TILE)
    valid = seg != -1
    validf = valid.astype(_f32)
    tgt = jnp.where(valid, seg, 0)
    row = jax.lax.broadcasted_iota(jnp.int32, logit.shape, 0)
    active = row < NUM_CLS
    neg = jnp.where(active, logit, -jnp.inf)
    m = jnp.max(neg, axis=0, keepdims=True)
    z = jnp.sum(jnp.where(active, jnp.exp(neg - m), 0.0),
                axis=0, keepdims=True)
    logz = jnp.log(z) + m
    ll = jnp.sum(jnp.where(row == tgt, logit, 0.0), axis=0, keepdims=True)
    nll = (logz - ll) * validf

    acc_ref[0:1, :] += nll
    acc_ref[1:2, :] += validf
    acc_ref[2:3, :] += bias_dist * mask
    acc_ref[3:4, :] += mask
    acc_ref[4:5, :] += cos * mask

    @pl.when(i == nsteps - 1)
    def _():
        seg_loss = jnp.sum(acc_ref[0, :]) / jnp.maximum(
            jnp.sum(acc_ref[1, :]), 1.0)
        denom = jnp.sum(acc_ref[3, :]) + 1e-8
        l1 = jnp.sum(acc_ref[2, :]) / denom
        cosl = jnp.sum(acc_ref[4, :]) / denom
        out_ref[0] = seg_loss + l1 + cosl
        out_ref[1] = seg_loss
        out_ref[2] = l1
        out_ref[3] = cosl


def kernel(coord, segment, instance, instance_center, bbox, offset, feat_in,
           W_bb, b_bb, W1, b1, gamma, beta, W2, b2, Ws, bs):
    S = pl.pallas_call(
        _gram_kernel,
        grid=(N // TILE_A,),
        in_specs=[pl.BlockSpec((TILE_A, 6), lambda i: (i, 0))],
        out_specs=pl.BlockSpec((8, 128), lambda i: (0, 0)),
        out_shape=jax.ShapeDtypeStruct((8, 128), _f32),
    )(feat_in)

    diff_src = jnp.concatenate(
        [coord.astype(_f32), instance_center.astype(_f32)], axis=1)

    WbbT = jnp.pad(W_bb.astype(_f32), ((0, 2), (0, 0))).T
    bbb_col = b_bb.astype(_f32)[:, None]
    W1T = W1.astype(_f32).T
    gbb = jnp.stack([gamma.astype(_f32), beta.astype(_f32),
                     b1.astype(_f32)], axis=1)
    gbb = jnp.pad(gbb, ((0, 0), (0, 5)))
    eye = jnp.eye(8, 3, dtype=_f32)
    W2T = jnp.pad(W2.astype(_f32), ((0, 0), (0, 5))).T
    b2_col = jnp.pad(b2.astype(_f32), (0, 5))[:, None]
    WsT = jnp.pad(Ws.astype(_f32), ((0, 0), (0, 12))).T
    bs_col = jnp.pad(bs.astype(_f32), (0, 12))[:, None]

    out = pl.pallas_call(
        _loss_kernel,
        grid=(N // TILE,),
        in_specs=[
            pl.BlockSpec((TILE, 6), lambda i: (i, 0)),
            pl.BlockSpec((TILE, 6), lambda i: (i, 0)),
            pl.BlockSpec((TILE,), lambda i: (i,)),
            pl.BlockSpec((TILE,), lambda i: (i,)),
            pl.BlockSpec((8, 128), lambda i: (0, 0)),
            pl.BlockSpec((256, 8), lambda i: (0, 0)),
            pl.BlockSpec((256, 1), lambda i: (0, 0)),
            pl.BlockSpec((256, 256), lambda i: (0, 0)),
            pl.BlockSpec((256, 8), lambda i: (0, 0)),
            pl.BlockSpec((8, 3), lambda i: (0, 0)),
            pl.BlockSpec((8, 256), lambda i: (0, 0)),
            pl.BlockSpec((8, 1), lambda i: (0, 0)),
            pl.BlockSpec((32, 256), lambda i: (0, 0)),
            pl.BlockSpec((32, 1), lambda i: (0, 0)),
        ],
        out_specs=pl.BlockSpec(memory_space=pltpu.SMEM),
        out_shape=jax.ShapeDtypeStruct((4,), _f32),
        scratch_shapes=[pltpu.VMEM((8, TILE), _f32),
                        pltpu.VMEM((C, 8), _f32)],
    )(feat_in, diff_src, segment, instance, S,
      WbbT, bbb_col, W1T, gbb, eye, W2T, b2_col, WsT, bs_col)

    return (out[0], out[1], out[2], out[3])


# speedup vs baseline: 1.8433x; 1.8433x over previous
import jax
import jax.numpy as jnp
from jax.experimental import pallas as pl
from jax.experimental.pallas import tpu as pltpu

N = 262144
C_IN = 6
C = 256
NUM_CLS = 20

TILE_A = 32768
TILE = 4096

_f32 = jnp.float32
_HIGH = jax.lax.Precision.HIGHEST


def _gram_kernel(x_ref, out_ref):
    i = pl.program_id(0)
    x = x_ref[...]
    g = jax.lax.dot_general(x, x, (((1,), (1,)), ((), ())),
                            preferred_element_type=_f32,
                            precision=_HIGH)
    g = jnp.pad(g, ((0, 0), (0, 120)))

    @pl.when(i == 0)
    def _():
        out_ref[...] = jnp.zeros_like(out_ref)

    out_ref[...] += g


def _loss_kernel(x_ref, cg_ref, s_ref, wbb_ref, w1_ref, gbb_ref,
                 w2_ref, b2_ref, ws_ref, bs_ref, out_ref, acc_ref, ab_ref):
    i = pl.program_id(0)
    nsteps = pl.num_programs(0)

    def dot(a, b, prec=None):
        return jax.lax.dot_general(a, b, (((1,), (0,)), ((), ())),
                                   preferred_element_type=_f32,
                                   precision=prec)

    @pl.when(i == 0)
    def _():
        acc_ref[...] = jnp.zeros_like(acc_ref)
        gam = gbb_ref[:, 0:1]
        bet = gbb_ref[:, 1:2]
        b1c = gbb_ref[:, 2:3]
        col = jax.lax.broadcasted_iota(jnp.int32, (C, 8), 1)
        w_aug = (dot(w1_ref[...], wbb_ref[...], _HIGH)
                 + jnp.where(col == 6, b1c, 0.0))
        s8 = s_ref[:, 0:8]
        nf = _f32(N)
        mu = dot(w_aug, s8[:, 6:7], _HIGH) / nf
        sw = dot(w_aug, s8, _HIGH)
        ex2 = jnp.sum(sw * w_aug, axis=1, keepdims=True) / nf
        var = ex2 - mu * mu
        sd = jnp.sqrt(var + 1e-3)
        a_col = gam / sd
        b_col = (b1c - mu) * a_col + bet
        ab_ref[:, 0:1] = a_col
        ab_ref[:, 1:2] = b_col

    x = x_ref[...]
    feat = dot(wbb_ref[...], x)
    h = dot(w1_ref[...], feat)
    hn = jnp.maximum(h * ab_ref[:, 0:1] + ab_ref[:, 1:2], 0.0)
    bp = dot(w2_ref[...], hn) + b2_ref[...]
    bp3 = bp[0:3, :]

    cg = cg_ref[...]
    bg3 = cg[3:6, :] - cg[0:3, :]
    inst = jax.lax.bitcast_convert_type(cg[7:8, :], jnp.int32)
    mask = (inst != -1).astype(_f32)

    bias_dist = jnp.sum(jnp.abs(bp3 - bg3), axis=0, keepdims=True)
    bp_n = jnp.sqrt(jnp.sum(bp3 * bp3, axis=0, keepdims=True))
    bg_n = jnp.sqrt(jnp.sum(bg3 * bg3, axis=0, keepdims=True))
    bpn = bp3 / (bp_n + 1e-8)
    bgn = bg3 / (bg_n + 1e-8)
    cos = -jnp.sum(bpn * bgn, axis=0, keepdims=True)

    logit = dot(ws_ref[...], feat) + bs_ref[...]
    seg = jax.lax.bitcast_convert_type(cg[6:7, :], jnp.int32)
    valid = seg != -1
    validf = valid.astype(_f32)
    tgt = jnp.where(valid, seg, 0)
    row = jax.lax.broadcasted_iota(jnp.int32, logit.shape, 0)
    active = row < NUM_CLS
    neg = jnp.where(active, logit, -jnp.inf)
    m = jnp.max(neg, axis=0, keepdims=True)
    z = jnp.sum(jnp.where(active, jnp.exp(neg - m), 0.0),
                axis=0, keepdims=True)
    logz = jnp.log(z) + m
    ll = jnp.sum(jnp.where(row == tgt, logit, 0.0), axis=0, keepdims=True)
    nll = (logz - ll) * validf

    acc_ref[0:1, :] += nll
    acc_ref[1:2, :] += validf
    acc_ref[2:3, :] += bias_dist * mask
    acc_ref[3:4, :] += mask
    acc_ref[4:5, :] += cos * mask

    @pl.when(i == nsteps - 1)
    def _():
        seg_loss = jnp.sum(acc_ref[0, :]) / jnp.maximum(
            jnp.sum(acc_ref[1, :]), 1.0)
        denom = jnp.sum(acc_ref[3, :]) + 1e-8
        l1 = jnp.sum(acc_ref[2, :]) / denom
        cosl = jnp.sum(acc_ref[4, :]) / denom
        out_ref[0] = seg_loss + l1 + cosl
        out_ref[1] = seg_loss
        out_ref[2] = l1
        out_ref[3] = cosl


def kernel(coord, segment, instance, instance_center, bbox, offset, feat_in,
           W_bb, b_bb, W1, b1, gamma, beta, W2, b2, Ws, bs):
    ones = jnp.ones((N, 1), _f32)
    zeros = jnp.zeros((N, 1), _f32)
    xT = jnp.concatenate([feat_in, ones, zeros], axis=1).T

    segf = jax.lax.bitcast_convert_type(segment, _f32)
    instf = jax.lax.bitcast_convert_type(instance, _f32)
    cg = jnp.stack([coord[:, 0], coord[:, 1], coord[:, 2],
                    instance_center[:, 0], instance_center[:, 1],
                    instance_center[:, 2], segf, instf], axis=0)

    S = pl.pallas_call(
        _gram_kernel,
        grid=(N // TILE_A,),
        in_specs=[pl.BlockSpec((8, TILE_A), lambda i: (0, i))],
        out_specs=pl.BlockSpec((8, 128), lambda i: (0, 0)),
        out_shape=jax.ShapeDtypeStruct((8, 128), _f32),
    )(xT)

    gbb = jnp.stack([gamma.astype(_f32), beta.astype(_f32),
                     b1.astype(_f32)], axis=1)
    gbb = jnp.pad(gbb, ((0, 0), (0, 5)))

    WbbT = jnp.pad(W_bb.astype(_f32), ((0, 2), (0, 0))).T
    WbbT = WbbT.at[:, 6].add(b_bb.astype(_f32))
    W1T = W1.astype(_f32).T
    W2T = jnp.pad(W2.astype(_f32), ((0, 0), (0, 5))).T
    b2_col = jnp.pad(b2.astype(_f32), (0, 5))[:, None]
    WsT = jnp.pad(Ws.astype(_f32), ((0, 0), (0, 12))).T
    bs_col = jnp.pad(bs.astype(_f32), (0, 12))[:, None]

    out = pl.pallas_call(
        _loss_kernel,
        grid=(N // TILE,),
        in_specs=[
            pl.BlockSpec((8, TILE), lambda i: (0, i)),
            pl.BlockSpec((8, TILE), lambda i: (0, i)),
            pl.BlockSpec((8, 128), lambda i: (0, 0)),
            pl.BlockSpec((256, 8), lambda i: (0, 0)),
            pl.BlockSpec((256, 256), lambda i: (0, 0)),
            pl.BlockSpec((256, 8), lambda i: (0, 0)),
            pl.BlockSpec((8, 256), lambda i: (0, 0)),
            pl.BlockSpec((8, 1), lambda i: (0, 0)),
            pl.BlockSpec((32, 256), lambda i: (0, 0)),
            pl.BlockSpec((32, 1), lambda i: (0, 0)),
        ],
        out_specs=pl.BlockSpec(memory_space=pltpu.SMEM),
        out_shape=jax.ShapeDtypeStruct((4,), _f32),
        scratch_shapes=[pltpu.VMEM((8, TILE), _f32),
                        pltpu.VMEM((C, 8), _f32)],
    )(xT, cg, S, WbbT, W1T, gbb, W2T, b2_col, WsT, bs_col)

    return (out[0], out[1], out[2], out[3])


# speedup vs baseline: 1.9256x; 1.0447x over previous
import jax
import jax.numpy as jnp
from jax.experimental import pallas as pl
from jax.experimental.pallas import tpu as pltpu

N = 262144
C_IN = 6
C = 256
NUM_CLS = 20

TILE_A = 65536
TILE = 8192

_f32 = jnp.float32
_HIGH = jax.lax.Precision.HIGHEST


def _gram_kernel(x_ref, out_ref):
    i = pl.program_id(0)
    x = x_ref[...]
    g = jax.lax.dot_general(x, x, (((1,), (1,)), ((), ())),
                            preferred_element_type=_f32,
                            precision=_HIGH)
    g = jnp.pad(g, ((0, 0), (0, 120)))

    @pl.when(i == 0)
    def _():
        out_ref[...] = jnp.zeros_like(out_ref)

    out_ref[...] += g


def _loss_kernel(x_ref, cg_ref, s_ref, wbb_ref, w1_ref, gbb_ref,
                 w2_ref, b2_ref, ws_ref, bs_ref, out_ref, acc_ref, ab_ref):
    i = pl.program_id(0)
    nsteps = pl.num_programs(0)

    def dot(a, b, prec=None):
        return jax.lax.dot_general(a, b, (((1,), (0,)), ((), ())),
                                   preferred_element_type=_f32,
                                   precision=prec)

    @pl.when(i == 0)
    def _():
        acc_ref[...] = jnp.zeros_like(acc_ref)
        gam = gbb_ref[:, 0:1]
        bet = gbb_ref[:, 1:2]
        b1c = gbb_ref[:, 2:3]
        col = jax.lax.broadcasted_iota(jnp.int32, (C, 8), 1)
        w_aug = (dot(w1_ref[...], wbb_ref[...], _HIGH)
                 + jnp.where(col == 6, b1c, 0.0))
        s8 = s_ref[:, 0:8]
        nf = _f32(N)
        mu = dot(w_aug, s8[:, 6:7], _HIGH) / nf
        sw = dot(w_aug, s8, _HIGH)
        ex2 = jnp.sum(sw * w_aug, axis=1, keepdims=True) / nf
        var = ex2 - mu * mu
        sd = jnp.sqrt(var + 1e-3)
        a_col = gam / sd
        b_col = (b1c - mu) * a_col + bet
        ab_ref[:, 0:1] = a_col
        ab_ref[:, 1:2] = b_col

    x = x_ref[...]
    feat = dot(wbb_ref[...], x)
    h = dot(w1_ref[...], feat)
    hn = jnp.maximum(h * ab_ref[:, 0:1] + ab_ref[:, 1:2], 0.0)
    bp = dot(w2_ref[...], hn) + b2_ref[...]
    bp3 = bp[0:3, :]

    cg = cg_ref[...]
    bg3 = cg[3:6, :] - cg[0:3, :]
    inst = jax.lax.bitcast_convert_type(cg[7:8, :], jnp.int32)
    mask = (inst != -1).astype(_f32)

    bias_dist = jnp.sum(jnp.abs(bp3 - bg3), axis=0, keepdims=True)
    bp_n = jnp.sqrt(jnp.sum(bp3 * bp3, axis=0, keepdims=True))
    bg_n = jnp.sqrt(jnp.sum(bg3 * bg3, axis=0, keepdims=True))
    bpn = bp3 / (bp_n + 1e-8)
    bgn = bg3 / (bg_n + 1e-8)
    cos = -jnp.sum(bpn * bgn, axis=0, keepdims=True)

    logit = dot(ws_ref[...], feat) + bs_ref[...]
    seg = jax.lax.bitcast_convert_type(cg[6:7, :], jnp.int32)
    valid = seg != -1
    validf = valid.astype(_f32)
    tgt = jnp.where(valid, seg, 0)
    row = jax.lax.broadcasted_iota(jnp.int32, logit.shape, 0)
    active = row < NUM_CLS
    neg = jnp.where(active, logit, -jnp.inf)
    m = jnp.max(neg, axis=0, keepdims=True)
    z = jnp.sum(jnp.where(active, jnp.exp(neg - m), 0.0),
                axis=0, keepdims=True)
    logz = jnp.log(z) + m
    ll = jnp.sum(jnp.where(row == tgt, logit, 0.0), axis=0, keepdims=True)
    nll = (logz - ll) * validf

    acc_ref[0:1, :] += nll
    acc_ref[1:2, :] += validf
    acc_ref[2:3, :] += bias_dist * mask
    acc_ref[3:4, :] += mask
    acc_ref[4:5, :] += cos * mask

    @pl.when(i == nsteps - 1)
    def _():
        seg_loss = jnp.sum(acc_ref[0, :]) / jnp.maximum(
            jnp.sum(acc_ref[1, :]), 1.0)
        denom = jnp.sum(acc_ref[3, :]) + 1e-8
        l1 = jnp.sum(acc_ref[2, :]) / denom
        cosl = jnp.sum(acc_ref[4, :]) / denom
        out_ref[0] = seg_loss + l1 + cosl
        out_ref[1] = seg_loss
        out_ref[2] = l1
        out_ref[3] = cosl


def kernel(coord, segment, instance, instance_center, bbox, offset, feat_in,
           W_bb, b_bb, W1, b1, gamma, beta, W2, b2, Ws, bs):
    ones = jnp.ones((N, 1), _f32)
    zeros = jnp.zeros((N, 1), _f32)
    xT = jnp.concatenate([feat_in, ones, zeros], axis=1).T

    segf = jax.lax.bitcast_convert_type(segment, _f32)
    instf = jax.lax.bitcast_convert_type(instance, _f32)
    cg = jnp.stack([coord[:, 0], coord[:, 1], coord[:, 2],
                    instance_center[:, 0], instance_center[:, 1],
                    instance_center[:, 2], segf, instf], axis=0)

    S = pl.pallas_call(
        _gram_kernel,
        grid=(N // TILE_A,),
        in_specs=[pl.BlockSpec((8, TILE_A), lambda i: (0, i))],
        out_specs=pl.BlockSpec((8, 128), lambda i: (0, 0)),
        out_shape=jax.ShapeDtypeStruct((8, 128), _f32),
    )(xT)

    gbb = jnp.stack([gamma.astype(_f32), beta.astype(_f32),
                     b1.astype(_f32)], axis=1)
    gbb = jnp.pad(gbb, ((0, 0), (0, 5)))

    WbbT = jnp.pad(W_bb.astype(_f32), ((0, 2), (0, 0))).T
    WbbT = WbbT.at[:, 6].add(b_bb.astype(_f32))
    W1T = W1.astype(_f32).T
    W2T = jnp.pad(W2.astype(_f32), ((0, 0), (0, 5))).T
    b2_col = jnp.pad(b2.astype(_f32), (0, 5))[:, None]
    WsT = jnp.pad(Ws.astype(_f32), ((0, 0), (0, 12))).T
    bs_col = jnp.pad(bs.astype(_f32), (0, 12))[:, None]

    out = pl.pallas_call(
        _loss_kernel,
        grid=(N // TILE,),
        in_specs=[
            pl.BlockSpec((8, TILE), lambda i: (0, i)),
            pl.BlockSpec((8, TILE), lambda i: (0, i)),
            pl.BlockSpec((8, 128), lambda i: (0, 0)),
            pl.BlockSpec((256, 8), lambda i: (0, 0)),
            pl.BlockSpec((256, 256), lambda i: (0, 0)),
            pl.BlockSpec((256, 8), lambda i: (0, 0)),
            pl.BlockSpec((8, 256), lambda i: (0, 0)),
            pl.BlockSpec((8, 1), lambda i: (0, 0)),
            pl.BlockSpec((32, 256), lambda i: (0, 0)),
            pl.BlockSpec((32, 1), lambda i: (0, 0)),
        ],
        out_specs=pl.BlockSpec(memory_space=pltpu.SMEM),
        out_shape=jax.ShapeDtypeStruct((4,), _f32),
        scratch_shapes=[pltpu.VMEM((8, TILE), _f32),
                        pltpu.VMEM((C, 8), _f32)],
    )(xT, cg, S, WbbT, W1T, gbb, W2T, b2_col, WsT, bs_col)

    return (out[0], out[1], out[2], out[3])
